# baseline (speedup 1.0000x reference)
import functools

import jax
import jax.numpy as jnp
from jax import lax
from jax.experimental import pallas as pl
from jax.experimental.pallas import tpu as pltpu
from jax.experimental.pallas import tpu_sc as plsc

_N = 16777216
_NC = 2
_NS = 16
_NW = _NC * _NS
_L = 16
_CHUNK = 28672

_N_SC = 14 * _NW * 16384
_N_TC = _N - _N_SC
_PER_W = _N_SC // _NW
_NCHUNK = _PER_W // _CHUNK
_VECS = _CHUNK // _L

_TC_BLK = 1 << 20
_TC_G = _N_TC // _TC_BLK


def _fnr_sc_body(x_hbm, t_hbm, out_hbm, xbuf, tbuf, obuf, sx0, sx1, st0, st1):
    wid = lax.axis_index("s") * _NC + lax.axis_index("c")
    base = wid * _PER_W
    sems_x = (sx0, sx1)
    sems_t = (st0, st1)

    def issue(i, b):
        off = pl.multiple_of(base + i * _CHUNK, _CHUNK)
        pltpu.make_async_copy(x_hbm.at[pl.ds(off, _CHUNK)], xbuf.at[b],
                              sems_x[b]).start()
        pltpu.make_async_copy(t_hbm.at[pl.ds(off, _CHUNK)], tbuf.at[b],
                              sems_t[b]).start()

    def wait_slot(b):
        pltpu.make_async_copy(x_hbm.at[pl.ds(0, _CHUNK)], xbuf.at[b],
                              sems_x[b]).wait()
        pltpu.make_async_copy(t_hbm.at[pl.ds(0, _CHUNK)], tbuf.at[b],
                              sems_t[b]).wait()

    def chunk_acc(b, carry):
        def body(j, carry):
            fn, pos = carry
            x = xbuf[b, pl.ds(j * _L, _L)]
            t = tbuf[b, pl.ds(j * _L, _L)]
            fn = fn + jnp.where(x < 0.5, t, 0)
            pos = pos + t
            return fn, pos

        return lax.fori_loop(0, _VECS, body, carry, unroll=8)

    issue(0, 0)
    zero = jnp.zeros((_L,), jnp.int32)

    def outer(g, carry):
        for b in range(2):
            i = g * 2 + b

            @pl.when(i + 1 < _NCHUNK)
            def _():
                issue(i + 1, 1 - b)

            wait_slot(b)
            carry = chunk_acc(b, carry)
        return carry

    carry = lax.fori_loop(0, _NCHUNK // 2, outer, (zero, zero))
    if _NCHUNK % 2:
        b_last = (_NCHUNK - 1) % 2
        wait_slot(b_last)
        carry = chunk_acc(b_last, carry)
    acc_fn, acc_pos = carry

    obuf[0, :] = acc_fn
    obuf[1, :] = acc_pos
    pltpu.sync_copy(obuf, out_hbm.at[wid])


_fnr_sc = functools.partial(
    pl.kernel,
    out_type=jax.ShapeDtypeStruct((_NW, 2, _L), jnp.int32),
    mesh=plsc.VectorSubcoreMesh(core_axis_name="c", subcore_axis_name="s"),
    scratch_types=[
        pltpu.VMEM((2, _CHUNK), jnp.float32),
        pltpu.VMEM((2, _CHUNK), jnp.int32),
        pltpu.VMEM((2, _L), jnp.int32),
        pltpu.SemaphoreType.DMA,
        pltpu.SemaphoreType.DMA,
        pltpu.SemaphoreType.DMA,
        pltpu.SemaphoreType.DMA,
    ],
)(_fnr_sc_body)


def _fnr_tc_body(x_ref, t_ref, o_ref):
    i = pl.program_id(0)

    @pl.when(i == 0)
    def _():
        o_ref[...] = jnp.zeros_like(o_ref)

    x = x_ref[...].reshape(-1, 8, 128)
    t = t_ref[...].reshape(-1, 8, 128)
    fnv = jnp.sum(jnp.where(x < 0.5, t, 0), axis=0)
    posv = jnp.sum(t, axis=0)
    o_ref[0] += fnv
    o_ref[1] += posv


_fnr_tc = pl.pallas_call(
    _fnr_tc_body,
    grid=(_TC_G,),
    in_specs=[
        pl.BlockSpec((_TC_BLK,), lambda i: (i + _N_SC // _TC_BLK,)),
        pl.BlockSpec((_TC_BLK,), lambda i: (i + _N_SC // _TC_BLK,)),
    ],
    out_specs=pl.BlockSpec((2, 8, 128), lambda i: (0, 0, 0)),
    out_shape=jax.ShapeDtypeStruct((2, 8, 128), jnp.int32),
)


@jax.jit
def kernel(inputs, targets):
    parts_sc = _fnr_sc(inputs, targets)
    parts_tc = _fnr_tc(inputs, targets)
    sums = parts_sc.sum(axis=(0, 2)) + parts_tc.sum(axis=(1, 2))
    fn = sums[0].astype(jnp.float32)
    pos = sums[1].astype(jnp.float32)
    return fn / jnp.maximum(pos, 1.0)


# speedup vs baseline: 1.0230x; 1.0230x over previous
import functools

import jax
import jax.numpy as jnp
from jax import lax
from jax.experimental import pallas as pl
from jax.experimental.pallas import tpu as pltpu
from jax.experimental.pallas import tpu_sc as plsc

_N = 16777216
_NC = 2
_NS = 16
_NW = _NC * _NS
_L = 16
_CHUNK = 16384

_N_SC = 14 * _NW * 16384
_N_TC = _N - _N_SC
_PER_W = _N_SC // _NW
_NCHUNK = _PER_W // _CHUNK
_VECS = _CHUNK // _L

_TC_BLK = 1 << 19
_TC_G = _N_TC // _TC_BLK


def _fnr_sc_body(x_hbm, t_hbm, out_hbm, xbuf, tbuf, obuf, sx0, sx1, st0, st1):
    wid = lax.axis_index("s") * _NC + lax.axis_index("c")
    base = wid * _PER_W
    sems_x = (sx0, sx1)
    sems_t = (st0, st1)

    def issue(i, b):
        off = pl.multiple_of(base + i * _CHUNK, _CHUNK)
        pltpu.make_async_copy(x_hbm.at[pl.ds(off, _CHUNK)], xbuf.at[b],
                              sems_x[b]).start()
        pltpu.make_async_copy(t_hbm.at[pl.ds(off, _CHUNK)], tbuf.at[b],
                              sems_t[b]).start()

    def wait_slot(b):
        pltpu.make_async_copy(x_hbm.at[pl.ds(0, _CHUNK)], xbuf.at[b],
                              sems_x[b]).wait()
        pltpu.make_async_copy(t_hbm.at[pl.ds(0, _CHUNK)], tbuf.at[b],
                              sems_t[b]).wait()

    def chunk_acc(b, carry):
        def body(j, carry):
            fn, pos = carry
            x = xbuf[b, pl.ds(j * _L, _L)]
            t = tbuf[b, pl.ds(j * _L, _L)]
            fn = fn + jnp.where(x < 0.5, t, 0)
            pos = pos + t
            return fn, pos

        return lax.fori_loop(0, _VECS, body, carry, unroll=4)

    issue(0, 0)
    zero = jnp.zeros((_L,), jnp.int32)

    def outer(g, carry):
        for b in range(2):
            i = g * 2 + b

            @pl.when(i + 1 < _NCHUNK)
            def _():
                issue(i + 1, 1 - b)

            wait_slot(b)
            carry = chunk_acc(b, carry)
        return carry

    carry = lax.fori_loop(0, _NCHUNK // 2, outer, (zero, zero))
    if _NCHUNK % 2:
        b_last = (_NCHUNK - 1) % 2
        wait_slot(b_last)
        carry = chunk_acc(b_last, carry)
    acc_fn, acc_pos = carry

    obuf[0, :] = acc_fn
    obuf[1, :] = acc_pos
    pltpu.sync_copy(obuf, out_hbm.at[wid])


_fnr_sc = functools.partial(
    pl.kernel,
    out_type=jax.ShapeDtypeStruct((_NW, 2, _L), jnp.int32),
    mesh=plsc.VectorSubcoreMesh(core_axis_name="c", subcore_axis_name="s"),
    scratch_types=[
        pltpu.VMEM((2, _CHUNK), jnp.float32),
        pltpu.VMEM((2, _CHUNK), jnp.int32),
        pltpu.VMEM((2, _L), jnp.int32),
        pltpu.SemaphoreType.DMA,
        pltpu.SemaphoreType.DMA,
        pltpu.SemaphoreType.DMA,
        pltpu.SemaphoreType.DMA,
    ],
)(_fnr_sc_body)


def _fnr_tc_body(x_ref, t_ref, o_ref):
    i = pl.program_id(0)

    @pl.when(i == 0)
    def _():
        o_ref[...] = jnp.zeros_like(o_ref)

    x = x_ref[...].reshape(-1, 8, 128)
    t = t_ref[...].reshape(-1, 8, 128)
    fnv = jnp.sum(jnp.where(x < 0.5, t, 0), axis=0)
    posv = jnp.sum(t, axis=0)
    o_ref[0] += fnv
    o_ref[1] += posv


_fnr_tc = pl.pallas_call(
    _fnr_tc_body,
    grid=(_TC_G,),
    in_specs=[
        pl.BlockSpec((_TC_BLK,), lambda i: (i + _N_SC // _TC_BLK,)),
        pl.BlockSpec((_TC_BLK,), lambda i: (i + _N_SC // _TC_BLK,)),
    ],
    out_specs=pl.BlockSpec((2, 8, 128), lambda i: (0, 0, 0)),
    out_shape=jax.ShapeDtypeStruct((2, 8, 128), jnp.int32),
)


@jax.jit
def kernel(inputs, targets):
    parts_sc = _fnr_sc(inputs, targets)
    parts_tc = _fnr_tc(inputs, targets)
    sums = parts_sc.sum(axis=(0, 2)) + parts_tc.sum(axis=(1, 2))
    fn = sums[0].astype(jnp.float32)
    pos = sums[1].astype(jnp.float32)
    return fn / jnp.maximum(pos, 1.0)
